# Initial kernel scaffold; baseline (speedup 1.0000x reference)
import jax
import jax.numpy as jnp
from jax.experimental import pallas as pl


def kernel(hidden_states, router_w1, router_b1, router_w2, router_b2, re_w1, re_b1, re_w2, re_b2, se_w1, se_b1, se_w2, se_b2, out_w, out_b):
    raise NotImplementedError("write your pallas kernel here")


# speedup vs baseline: 2.6428x; 2.6428x over previous
import functools

import jax
import jax.numpy as jnp
from jax.experimental import pallas as pl

H = 768
D_FF = 3072
S = 2048
NR, NS, TOPK = 4, 2, 2
NE = NR + NS
FBLK = 768
NF = D_FF // FBLK
NEG = -1e30


def _gelu(x):
    return 0.5 * x * (1.0 + jax.lax.erf(x * 0.7071067811865476))


def _router_body(x_ref, w1_ref, b1_ref, w2_ref, b2_ref, wmat_ref, z_ref):
    x = x_ref[...]
    hr = _gelu(jnp.dot(x, w1_ref[...], preferred_element_type=jnp.float32)
               + b1_ref[...])
    logits = (jnp.dot(hr, w2_ref[...], preferred_element_type=jnp.float32)
              + b2_ref[...])
    col = jax.lax.broadcasted_iota(jnp.int32, logits.shape, 1)
    logits = jnp.where(col < NR, logits, NEG)
    m1 = jnp.max(logits, axis=-1, keepdims=True)
    idx1 = jnp.min(jnp.where(logits == m1, col, 1000), axis=-1, keepdims=True)
    l2 = jnp.where(col == idx1, NEG, logits)
    m2 = jnp.max(l2, axis=-1, keepdims=True)
    idx2 = jnp.min(jnp.where(l2 == m2, col, 1000), axis=-1, keepdims=True)
    e2 = jnp.exp(m2 - m1)
    wa = 1.0 / (1.0 + e2)
    wb = e2 * wa
    for j in range(NR):
        row = (jnp.where(idx1[:, 0] == j, wa[:, 0], 0.0)
               + jnp.where(idx2[:, 0] == j, wb[:, 0], 0.0))
        wmat_ref[j, :] = row
    half = jnp.full((S,), 1.0 / NS, jnp.float32)
    for j in range(NS):
        wmat_ref[NR + j, :] = half
    zero = jnp.zeros((S,), jnp.float32)
    for j in range(NR + NS, 8):
        wmat_ref[j, :] = zero
    lse = m1[:, 0] + jnp.log(jnp.sum(jnp.exp(logits - m1), axis=-1))
    z_ref[...] = jnp.mean(jnp.square(lse)).reshape(1, 1)


def _expert_body(x_ref, rw1_ref, rb1_ref, rw2_ref, rb2_ref,
                 sw1_ref, sb1_ref, sw2_ref, sb2_ref, wmat_ref, acc_ref):
    e = pl.program_id(0)
    f = pl.program_id(1)

    @pl.when(jnp.logical_and(e == 0, f == 0))
    def _init():
        acc_ref[...] = jnp.zeros_like(acc_ref)

    routed = e < NR
    w1 = jnp.where(routed, rw1_ref[0], sw1_ref[0])
    w2 = jnp.where(routed, rw2_ref[0], sw2_ref[0])
    b1 = jnp.where(routed, rb1_ref[0, 0], sb1_ref[0, 0])
    x = x_ref[...]
    h = _gelu(jnp.dot(x, w1, preferred_element_type=jnp.float32) + b1)
    contrib = jnp.dot(h, w2, preferred_element_type=jnp.float32)
    wcol = jnp.zeros((S,), jnp.float32)
    for j in range(NE):
        wcol = wcol + jnp.where(e == j, wmat_ref[j, :], 0.0)

    @pl.when(f == 0)
    def _bias():
        b2 = jnp.where(routed, rb2_ref[0, 0], sb2_ref[0, 0])
        acc_ref[...] += wcol[:, None] * b2[None, :]

    acc_ref[...] += wcol[:, None] * contrib


def _finish_body(acc_ref, x_ref, w_ref, b_ref, out_ref):
    o = jnp.dot(acc_ref[...], w_ref[...], preferred_element_type=jnp.float32)
    o = (o + b_ref[...]) * 0.5 + x_ref[...] * 0.5
    mean = jnp.mean(o, axis=-1, keepdims=True)
    o = o - mean
    var = jnp.mean(jnp.square(o), axis=-1, keepdims=True)
    out_ref[...] = o * jax.lax.rsqrt(var + 1e-6)


def _const_spec(shape):
    return pl.BlockSpec(shape, lambda *_: tuple(0 for _ in shape))


@functools.partial(jax.jit, static_argnames=("interpret",))
def _run(x2d, router_w1, router_b1, router_w2p, router_b2p,
         re_w1, re_b1, re_w2, re_b2,
         se_w1, se_b1, se_w2, se_b2, out_w, out_b, interpret=False):
    wmat, z_loss = pl.pallas_call(
        _router_body,
        grid=(1,),
        in_specs=[_const_spec((S, H)), _const_spec((H, H)),
                  _const_spec((1, H)), _const_spec((H, 8)),
                  _const_spec((1, 8))],
        out_specs=[_const_spec((8, S)), _const_spec((1, 1))],
        out_shape=[jax.ShapeDtypeStruct((8, S), jnp.float32),
                   jax.ShapeDtypeStruct((1, 1), jnp.float32)],
        interpret=interpret,
    )(x2d, router_w1, router_b1.reshape(1, H), router_w2p, router_b2p)

    def re_w1_idx(e, f):
        return (jnp.minimum(e, NR - 1), 0, jnp.where(e < NR, f, NF - 1))

    def se_w1_idx(e, f):
        return (jnp.clip(e - NR, 0, NS - 1), 0, jnp.where(e < NR, 0, f))

    def re_w2_idx(e, f):
        return (jnp.minimum(e, NR - 1), jnp.where(e < NR, f, NF - 1), 0)

    def se_w2_idx(e, f):
        return (jnp.clip(e - NR, 0, NS - 1), jnp.where(e < NR, 0, f), 0)

    acc = pl.pallas_call(
        _expert_body,
        grid=(NE, NF),
        in_specs=[
            _const_spec((S, H)),
            pl.BlockSpec((1, H, FBLK), re_w1_idx),
            pl.BlockSpec((1, 1, FBLK),
                         lambda e, f: (jnp.minimum(e, NR - 1), 0,
                                       jnp.where(e < NR, f, NF - 1))),
            pl.BlockSpec((1, FBLK, H), re_w2_idx),
            pl.BlockSpec((1, 1, H), lambda e, f: (jnp.minimum(e, NR - 1), 0, 0)),
            pl.BlockSpec((1, H, FBLK), se_w1_idx),
            pl.BlockSpec((1, 1, FBLK),
                         lambda e, f: (jnp.clip(e - NR, 0, NS - 1), 0,
                                       jnp.where(e < NR, 0, f))),
            pl.BlockSpec((1, FBLK, H), se_w2_idx),
            pl.BlockSpec((1, 1, H), lambda e, f: (jnp.clip(e - NR, 0, NS - 1), 0, 0)),
            _const_spec((8, S)),
        ],
        out_specs=_const_spec((S, H)),
        out_shape=jax.ShapeDtypeStruct((S, H), jnp.float32),
        interpret=interpret,
    )(x2d, re_w1, re_b1.reshape(NR, 1, D_FF), re_w2, re_b2.reshape(NR, 1, H),
      se_w1, se_b1.reshape(NS, 1, D_FF), se_w2, se_b2.reshape(NS, 1, H), wmat)

    out = pl.pallas_call(
        _finish_body,
        grid=(1,),
        in_specs=[_const_spec((S, H)), _const_spec((S, H)),
                  _const_spec((H, H)), _const_spec((1, H))],
        out_specs=_const_spec((S, H)),
        out_shape=jax.ShapeDtypeStruct((S, H), jnp.float32),
        interpret=interpret,
    )(acc, x2d, out_w, out_b.reshape(1, H))
    return out, z_loss


def kernel(hidden_states, router_w1, router_b1, router_w2, router_b2,
           re_w1, re_b1, re_w2, re_b2, se_w1, se_b1, se_w2, se_b2,
           out_w, out_b, interpret=False):
    x2d = hidden_states.reshape(S, H)
    router_w2p = jnp.pad(router_w2, ((0, 0), (0, 8 - NR)))
    router_b2p = jnp.pad(router_b2, (0, 8 - NR)).reshape(1, 8)
    out, z_loss = _run(x2d, router_w1, router_b1, router_w2p, router_b2p,
                       re_w1, re_b1, re_w2, re_b2,
                       se_w1, se_b1, se_w2, se_b2, out_w, out_b,
                       interpret=interpret)
    return out.reshape(1, S, H), z_loss[0, 0]
